# Initial kernel scaffold; baseline (speedup 1.0000x reference)
import jax
import jax.numpy as jnp
from jax.experimental import pallas as pl


def kernel(logits, xyz, rgb):
    raise NotImplementedError("write your pallas kernel here")


# speedup vs baseline: 10.0896x; 10.0896x over previous
import jax
import jax.numpy as jnp
from jax import lax
from jax.experimental import pallas as pl
from jax.experimental.pallas import tpu as pltpu

K = 16
N = 4096
C = 13
QB = 256


def _table_kernel(logits_ref, rgb_ref, table_ref):
    x = logits_ref[...]
    m = jnp.max(x, axis=1, keepdims=True)
    e = jnp.exp(x - m)
    s = jnp.sum(e, axis=1, keepdims=True)
    table_ref[:, 0:3] = rgb_ref[...]
    table_ref[:, 3:16] = e / s


def _loss_kernel(xyzq_ref, xyzall_ref, table_ref, cent_ref, out_ref):
    b = pl.program_id(0)
    i = pl.program_id(1)
    xq = xyzq_ref[...]
    dist = None
    for d in range(3):
        a = xq[:, d : d + 1]
        row = xyzall_ref[d : d + 1, :]
        t = (a - row) ** 2
        dist = t if dist is None else dist + t
    iota = lax.broadcasted_iota(jnp.int32, (QB, N), 1)
    key = jnp.bitwise_or(
        jnp.bitwise_and(lax.bitcast_convert_type(dist, jnp.int32),
                        jnp.int32(-4096)),
        iota,
    )
    table = table_ref[...]
    table_hi = table.astype(jnp.bfloat16).astype(jnp.float32)
    table_lo = table - table_hi
    cent = cent_ref[...]
    crgb = cent[:, 0:3]
    cp = cent[:, 3:16]
    acc = jnp.zeros((), jnp.float32)
    for _ in range(K):
        kmin = jnp.min(key, axis=1, keepdims=True)
        onehot = key == kmin
        key = jnp.where(onehot, jnp.int32(0x7FFFFFFF), key)
        ohf = onehot.astype(jnp.float32)
        feat = (jnp.dot(ohf, table_hi, preferred_element_type=jnp.float32)
                + jnp.dot(ohf, table_lo, preferred_element_type=jnp.float32))
        nrgb = feat[:, 0:3]
        npb = feat[:, 3:16]
        d2 = jnp.sum((crgb - nrgb) ** 2, axis=1) + 1e-12
        w = jnp.exp(-10.0 * jnp.sqrt(d2))
        pd = jnp.sum(jnp.abs(cp - npb), axis=1)
        acc += jnp.sum(w * pd)

    @pl.when((b == 0) & (i == 0))
    def _():
        out_ref[0, 0] = 0.0

    out_ref[0, 0] += acc


def kernel(logits, xyz, rgb):
    B = logits.shape[0]
    logitsT = jnp.transpose(logits, (0, 2, 1))
    rgbT = jnp.transpose(rgb, (0, 2, 1))
    xyzT = jnp.transpose(xyz, (0, 2, 1))

    table = pl.pallas_call(
        _table_kernel,
        grid=(B,),
        in_specs=[
            pl.BlockSpec((None, N, C), lambda b: (b, 0, 0)),
            pl.BlockSpec((None, N, 3), lambda b: (b, 0, 0)),
        ],
        out_shape=jax.ShapeDtypeStruct((B, N, 16), jnp.float32),
        out_specs=pl.BlockSpec((None, N, 16), lambda b: (b, 0, 0)),
    )(logitsT, rgbT)

    nsteps = N // QB
    loss_sum = pl.pallas_call(
        _loss_kernel,
        grid=(B, nsteps),
        in_specs=[
            pl.BlockSpec((None, QB, 3), lambda b, i: (b, i, 0)),
            pl.BlockSpec((None, 3, N), lambda b, i: (b, 0, 0)),
            pl.BlockSpec((None, N, 16), lambda b, i: (b, 0, 0)),
            pl.BlockSpec((None, QB, 16), lambda b, i: (b, i, 0)),
        ],
        out_shape=jax.ShapeDtypeStruct((1, 1), jnp.float32),
        out_specs=pl.BlockSpec(memory_space=pltpu.SMEM),
    )(xyzT, xyz, table, table)

    return loss_sum[0, 0] / (B * N * K)


# speedup vs baseline: 20.0306x; 1.9853x over previous
import functools

import jax
import jax.numpy as jnp
from jax import lax
from jax.experimental import pallas as pl
from jax.experimental.pallas import tpu as pltpu
from jax.experimental.pallas import tpu_sc as plsc

K = 16
N = 4096
C = 13
QB = 256
NW = 32
QW = 256


def _table_kernel(logits_ref, rgb_ref, table_ref):
    x = logits_ref[...]
    m = jnp.max(x, axis=0, keepdims=True)
    e = jnp.exp(x - m)
    s = jnp.sum(e, axis=0, keepdims=True)
    table_ref[0:3, :] = rgb_ref[...]
    table_ref[3:16, :] = e / s


def _topk_kernel(xyzq_ref, xyzall_ref, idx_ref):
    xq = xyzq_ref[...]
    dist = None
    for d in range(3):
        a = xq[:, d : d + 1]
        row = xyzall_ref[d : d + 1, :]
        t = (a - row) ** 2
        dist = t if dist is None else dist + t
    iota = lax.broadcasted_iota(jnp.int32, (QB, N), 1)
    key = jnp.bitwise_or(
        jnp.bitwise_and(lax.bitcast_convert_type(dist, jnp.int32),
                        jnp.int32(-4096)),
        iota,
    )
    cols = []
    for _ in range(K):
        kmin = jnp.min(key, axis=1, keepdims=True)
        key = jnp.where(key == kmin, jnp.int32(0x7FFFFFFF), key)
        cols.append(jnp.bitwise_and(kmin, jnp.int32(0xFFF)))
    idx_ref[...] = jnp.concatenate(cols, axis=1)


def _sc_combine_body(table_hbm, idx_hbm, out_hbm, tab_v, idx_v, acc_v):
    wid = lax.axis_index("s") * 2 + lax.axis_index("c")
    b = wid // 16
    base = (wid % 16) * QW
    pltpu.sync_copy(table_hbm.at[b], tab_v)
    pltpu.sync_copy(idx_hbm.at[b, pl.ds(base * K, QW * K)], idx_v)
    iota16 = lax.broadcasted_iota(jnp.int32, (16,), 0)

    def body_qb(qb, acc):
        cent = [tab_v[pl.ds(c * N + base + qb * 16, 16)] for c in range(16)]

        def body_k(k, acc):
            ivec = plsc.load_gather(idx_v, [qb * 256 + k + iota16 * K])
            g = [plsc.load_gather(tab_v, [ivec + c * N]) for c in range(16)]
            dr = g[0] - cent[0]
            dg = g[1] - cent[1]
            db = g[2] - cent[2]
            d2 = dr * dr + dg * dg + db * db + 1e-12
            i = lax.bitcast_convert_type(d2, jnp.int32)
            i = jnp.int32(0x5F3759DF) - lax.shift_right_arithmetic(i, 1)
            y = lax.bitcast_convert_type(i, jnp.float32)
            for _ in range(3):
                y = y * (1.5 - 0.5 * d2 * y * y)
            rd = d2 * y
            w = jnp.exp(-10.0 * rd)
            pd = None
            for c in range(3, 16):
                t = jnp.abs(g[c] - cent[c])
                pd = t if pd is None else pd + t
            return acc + w * pd

        return lax.fori_loop(0, 16, body_k, acc)

    acc = lax.fori_loop(0, 16, body_qb, jnp.zeros((16,), jnp.float32))
    acc_v[...] = acc
    pltpu.sync_copy(acc_v, out_hbm.at[wid])


_sc_combine = functools.partial(
    pl.kernel,
    mesh=plsc.VectorSubcoreMesh(core_axis_name="c", subcore_axis_name="s"),
    out_type=jax.ShapeDtypeStruct((NW, 16), jnp.float32),
    scratch_types=[
        pltpu.VMEM((16 * N,), jnp.float32),
        pltpu.VMEM((QW * K,), jnp.int32),
        pltpu.VMEM((16,), jnp.float32),
    ],
    compiler_params=pltpu.CompilerParams(needs_layout_passes=False),
)(_sc_combine_body)


def kernel(logits, xyz, rgb):
    B = logits.shape[0]
    xyzT = jnp.transpose(xyz, (0, 2, 1))

    table = pl.pallas_call(
        _table_kernel,
        grid=(B,),
        in_specs=[
            pl.BlockSpec((None, C, N), lambda b: (b, 0, 0)),
            pl.BlockSpec((None, 3, N), lambda b: (b, 0, 0)),
        ],
        out_shape=jax.ShapeDtypeStruct((B, 16, N), jnp.float32),
        out_specs=pl.BlockSpec((None, 16, N), lambda b: (b, 0, 0)),
    )(logits, rgb)

    nsteps = N // QB
    idx = pl.pallas_call(
        _topk_kernel,
        grid=(B, nsteps),
        in_specs=[
            pl.BlockSpec((None, QB, 3), lambda b, i: (b, i, 0)),
            pl.BlockSpec((None, 3, N), lambda b, i: (b, 0, 0)),
        ],
        out_shape=jax.ShapeDtypeStruct((B, N, K), jnp.int32),
        out_specs=pl.BlockSpec((None, QB, K), lambda b, i: (b, i, 0)),
    )(xyzT, xyz)

    partials = _sc_combine(table.reshape(B, 16 * N),
                           idx.reshape(B, N * K))
    return jnp.sum(partials) / (B * N * K)


# speedup vs baseline: 31.3909x; 1.5671x over previous
import functools

import jax
import jax.numpy as jnp
from jax import lax
from jax.experimental import pallas as pl
from jax.experimental.pallas import tpu as pltpu
from jax.experimental.pallas import tpu_sc as plsc

K = 16
N = 4096
C = 13
QB = 256
NW = 32
QW = 256


def _table_kernel(logits_ref, rgb_ref, table_ref):
    x = logits_ref[...]
    m = jnp.max(x, axis=0, keepdims=True)
    e = jnp.exp(x - m)
    s = jnp.sum(e, axis=0, keepdims=True)
    table_ref[0:3, :] = rgb_ref[...]
    table_ref[3:16, :] = e / s


R_ROUNDS = 5
NLB = N // 128


def _topk_kernel(xyzq_ref, xyzall_ref, idx_ref):
    xq = xyzq_ref[...]
    dist = None
    for d in range(3):
        a = xq[:, d : d + 1]
        row = xyzall_ref[d : d + 1, :]
        t = (a - row) ** 2
        dist = t if dist is None else dist + t
    iota = lax.broadcasted_iota(jnp.int32, (QB, N), 1)
    key = jnp.bitwise_or(
        jnp.bitwise_and(lax.bitcast_convert_type(dist, jnp.int32),
                        jnp.int32(-4096)),
        iota,
    )
    cands = []
    for r in range(R_ROUNDS):
        colmin = key[:, 0:128]
        for c in range(1, NLB):
            colmin = jnp.minimum(colmin, key[:, c * 128 : (c + 1) * 128])
        cands.append(colmin)
        if r + 1 < R_ROUNDS:
            big = jnp.concatenate([colmin] * NLB, axis=1)
            key = jnp.where(key == big, jnp.int32(0x7FFFFFFF), key)
    cand = jnp.concatenate(cands, axis=1)
    cols = []
    for _ in range(K):
        kmin = jnp.min(cand, axis=1, keepdims=True)
        cand = jnp.where(cand == kmin, jnp.int32(0x7FFFFFFF), cand)
        cols.append(jnp.bitwise_and(kmin, jnp.int32(0xFFF)))
    idx_ref[...] = jnp.concatenate(cols, axis=1)


def _sc_combine_body(table_hbm, idx_hbm, out_hbm, tab_v, idx_v, acc_v):
    wid = lax.axis_index("s") * 2 + lax.axis_index("c")
    b = wid // 16
    base = (wid % 16) * QW
    pltpu.sync_copy(table_hbm.at[b], tab_v)
    pltpu.sync_copy(idx_hbm.at[b, pl.ds(base * K, QW * K)], idx_v)
    iota16 = lax.broadcasted_iota(jnp.int32, (16,), 0)

    def body_qb(qb, acc):
        cent = [tab_v[pl.ds(c * N + base + qb * 16, 16)] for c in range(16)]

        def body_k(k, acc):
            ivec = plsc.load_gather(idx_v, [qb * 256 + k + iota16 * K])
            g = [plsc.load_gather(tab_v, [ivec + c * N]) for c in range(16)]
            dr = g[0] - cent[0]
            dg = g[1] - cent[1]
            db = g[2] - cent[2]
            d2 = dr * dr + dg * dg + db * db + 1e-12
            i = lax.bitcast_convert_type(d2, jnp.int32)
            i = jnp.int32(0x5F3759DF) - lax.shift_right_arithmetic(i, 1)
            y = lax.bitcast_convert_type(i, jnp.float32)
            for _ in range(3):
                y = y * (1.5 - 0.5 * d2 * y * y)
            rd = d2 * y
            w = jnp.exp(-10.0 * rd)
            pd = None
            for c in range(3, 16):
                t = jnp.abs(g[c] - cent[c])
                pd = t if pd is None else pd + t
            return acc + w * pd

        return lax.fori_loop(0, 16, body_k, acc)

    acc = lax.fori_loop(0, 16, body_qb, jnp.zeros((16,), jnp.float32))
    acc_v[...] = acc
    pltpu.sync_copy(acc_v, out_hbm.at[wid])


@functools.lru_cache(maxsize=1)
def _get_sc_combine():
    return functools.partial(
        pl.kernel,
        mesh=plsc.VectorSubcoreMesh(core_axis_name="c", subcore_axis_name="s"),
        out_type=jax.ShapeDtypeStruct((NW, 16), jnp.float32),
        scratch_types=[
            pltpu.VMEM((16 * N,), jnp.float32),
            pltpu.VMEM((QW * K,), jnp.int32),
            pltpu.VMEM((16,), jnp.float32),
        ],
        compiler_params=pltpu.CompilerParams(needs_layout_passes=False),
    )(_sc_combine_body)


def kernel(logits, xyz, rgb):
    B = logits.shape[0]
    xyzT = jnp.transpose(xyz, (0, 2, 1))

    table = pl.pallas_call(
        _table_kernel,
        grid=(B,),
        in_specs=[
            pl.BlockSpec((None, C, N), lambda b: (b, 0, 0)),
            pl.BlockSpec((None, 3, N), lambda b: (b, 0, 0)),
        ],
        out_shape=jax.ShapeDtypeStruct((B, 16, N), jnp.float32),
        out_specs=pl.BlockSpec((None, 16, N), lambda b: (b, 0, 0)),
    )(logits, rgb)

    nsteps = N // QB
    idx = pl.pallas_call(
        _topk_kernel,
        grid=(B, nsteps),
        in_specs=[
            pl.BlockSpec((None, QB, 3), lambda b, i: (b, i, 0)),
            pl.BlockSpec((None, 3, N), lambda b, i: (b, 0, 0)),
        ],
        out_shape=jax.ShapeDtypeStruct((B, N, K), jnp.int32),
        out_specs=pl.BlockSpec((None, QB, K), lambda b, i: (b, i, 0)),
    )(xyzT, xyz)

    partials = _get_sc_combine()(table.reshape(B, 16 * N),
                                 idx.reshape(B, N * K))
    return jnp.sum(partials) / (B * N * K)


# speedup vs baseline: 32.4540x; 1.0339x over previous
import functools

import jax
import jax.numpy as jnp
from jax import lax
from jax.experimental import pallas as pl
from jax.experimental.pallas import tpu as pltpu
from jax.experimental.pallas import tpu_sc as plsc

K = 16
N = 4096
C = 13
QB = 256
NW = 32
QW = 256


def _table_kernel(logits_ref, rgb_ref, table_ref):
    x = logits_ref[...]
    m = jnp.max(x, axis=0, keepdims=True)
    e = jnp.exp(x - m)
    s = jnp.sum(e, axis=0, keepdims=True)
    table_ref[0:3, :] = rgb_ref[...]
    table_ref[3:16, :] = e / s


R_ROUNDS = 5
NLB = N // 128


def _topk_kernel(xyzq_ref, xyzall_ref, idx_ref):
    xq = xyzq_ref[...]
    xall = xyzall_ref[...]
    ab = jnp.dot(xq.astype(jnp.bfloat16), xall.astype(jnp.bfloat16),
                 preferred_element_type=jnp.float32)
    xxq = jnp.sum(xq * xq, axis=1, keepdims=True)
    xxa = jnp.sum(xall * xall, axis=0, keepdims=True)
    dist = (xxq + (-2.0) * ab) + xxa
    dist = jnp.maximum(dist, 0.0)
    iota = lax.broadcasted_iota(jnp.int32, (QB, N), 1)
    key = jnp.bitwise_or(
        jnp.bitwise_and(lax.bitcast_convert_type(dist, jnp.int32),
                        jnp.int32(-4096)),
        iota,
    )
    cands = []
    for r in range(R_ROUNDS):
        colmin = key[:, 0:128]
        for c in range(1, NLB):
            colmin = jnp.minimum(colmin, key[:, c * 128 : (c + 1) * 128])
        cands.append(colmin)
        if r + 1 < R_ROUNDS:
            big = jnp.concatenate([colmin] * NLB, axis=1)
            key = jnp.where(key == big, jnp.int32(0x7FFFFFFF), key)
    cand = jnp.concatenate(cands, axis=1)
    cols = []
    for _ in range(K):
        kmin = jnp.min(cand, axis=1, keepdims=True)
        cand = jnp.where(cand == kmin, jnp.int32(0x7FFFFFFF), cand)
        cols.append(jnp.bitwise_and(kmin, jnp.int32(0xFFF)))
    idx_ref[...] = jnp.concatenate(cols, axis=1)


def _sc_combine_body(table_hbm, idx_hbm, out_hbm, tab_v, idx_v, acc_v):
    wid = lax.axis_index("s") * 2 + lax.axis_index("c")
    b = wid // 16
    base = (wid % 16) * QW
    pltpu.sync_copy(table_hbm.at[b], tab_v)
    pltpu.sync_copy(idx_hbm.at[b, pl.ds(base * K, QW * K)], idx_v)
    iota16 = lax.broadcasted_iota(jnp.int32, (16,), 0)

    def body_qb(qb, acc):
        cent = [tab_v[pl.ds(c * N + base + qb * 16, 16)] for c in range(16)]

        def body_k(k, acc):
            ivec = plsc.load_gather(idx_v, [qb * 256 + k + iota16 * K])
            g = [plsc.load_gather(tab_v, [ivec + c * N]) for c in range(16)]
            dr = g[0] - cent[0]
            dg = g[1] - cent[1]
            db = g[2] - cent[2]
            d2 = dr * dr + dg * dg + db * db + 1e-12
            i = lax.bitcast_convert_type(d2, jnp.int32)
            i = jnp.int32(0x5F3759DF) - lax.shift_right_arithmetic(i, 1)
            y = lax.bitcast_convert_type(i, jnp.float32)
            for _ in range(3):
                y = y * (1.5 - 0.5 * d2 * y * y)
            rd = d2 * y
            w = jnp.exp(-10.0 * rd)
            pd = None
            for c in range(3, 16):
                t = jnp.abs(g[c] - cent[c])
                pd = t if pd is None else pd + t
            return acc + w * pd

        return lax.fori_loop(0, 16, body_k, acc)

    acc = lax.fori_loop(0, 16, body_qb, jnp.zeros((16,), jnp.float32))
    acc_v[...] = acc
    pltpu.sync_copy(acc_v, out_hbm.at[wid])


@functools.lru_cache(maxsize=1)
def _get_sc_combine():
    return functools.partial(
        pl.kernel,
        mesh=plsc.VectorSubcoreMesh(core_axis_name="c", subcore_axis_name="s"),
        out_type=jax.ShapeDtypeStruct((NW, 16), jnp.float32),
        scratch_types=[
            pltpu.VMEM((16 * N,), jnp.float32),
            pltpu.VMEM((QW * K,), jnp.int32),
            pltpu.VMEM((16,), jnp.float32),
        ],
        compiler_params=pltpu.CompilerParams(needs_layout_passes=False),
    )(_sc_combine_body)


def kernel(logits, xyz, rgb):
    B = logits.shape[0]
    xyzT = jnp.transpose(xyz, (0, 2, 1))

    table = pl.pallas_call(
        _table_kernel,
        grid=(B,),
        in_specs=[
            pl.BlockSpec((None, C, N), lambda b: (b, 0, 0)),
            pl.BlockSpec((None, 3, N), lambda b: (b, 0, 0)),
        ],
        out_shape=jax.ShapeDtypeStruct((B, 16, N), jnp.float32),
        out_specs=pl.BlockSpec((None, 16, N), lambda b: (b, 0, 0)),
    )(logits, rgb)

    nsteps = N // QB
    idx = pl.pallas_call(
        _topk_kernel,
        grid=(B, nsteps),
        in_specs=[
            pl.BlockSpec((None, QB, 3), lambda b, i: (b, i, 0)),
            pl.BlockSpec((None, 3, N), lambda b, i: (b, 0, 0)),
        ],
        out_shape=jax.ShapeDtypeStruct((B, N, K), jnp.int32),
        out_specs=pl.BlockSpec((None, QB, K), lambda b, i: (b, i, 0)),
    )(xyzT, xyz)

    partials = _get_sc_combine()(table.reshape(B, 16 * N),
                                 idx.reshape(B, N * K))
    return jnp.sum(partials) / (B * N * K)


# speedup vs baseline: 37.5971x; 1.1585x over previous
import functools

import jax
import jax.numpy as jnp
from jax import lax
from jax.experimental import pallas as pl
from jax.experimental.pallas import tpu as pltpu
from jax.experimental.pallas import tpu_sc as plsc

K = 16
N = 4096
C = 13
QB = 256
NW = 32
QW = 256


def _table_kernel(logits_ref, rgb_ref, table_ref):
    x = logits_ref[...]
    m = jnp.max(x, axis=0, keepdims=True)
    e = jnp.exp(x - m)
    s = jnp.sum(e, axis=0, keepdims=True)
    table_ref[0:3, :] = rgb_ref[...]
    table_ref[3:16, :] = e / s


R_DEPTH = 4
NLB = N // 128


def _topk_kernel(xyzq_ref, xyzall_ref, idx_ref):
    xq = xyzq_ref[...]
    xall = xyzall_ref[...]
    ab = jnp.dot(xq.astype(jnp.bfloat16), xall.astype(jnp.bfloat16),
                 preferred_element_type=jnp.float32)
    xxq = jnp.sum(xq * xq, axis=1, keepdims=True)
    xxa = jnp.sum(xall * xall, axis=0, keepdims=True)
    inf = jnp.int32(0x7FFFFFFF)
    iota = lax.broadcasted_iota(jnp.int32, (QB, 128), 1)
    stacks = [jnp.full((QB, 128), inf, jnp.int32) for _ in range(R_DEPTH)]
    for c in range(NLB):
        sl = slice(c * 128, (c + 1) * 128)
        d = (xxq + (-2.0) * ab[:, sl]) + xxa[:, sl]
        d = jnp.maximum(d, 0.0)
        v = jnp.bitwise_or(
            jnp.bitwise_and(lax.bitcast_convert_type(d, jnp.int32),
                            jnp.int32(-4096)),
            iota + c * 128,
        )
        for r in range(R_DEPTH):
            lo = jnp.minimum(stacks[r], v)
            v = jnp.maximum(stacks[r], v)
            stacks[r] = lo
    cols = []
    for _ in range(K):
        kmin = jnp.min(stacks[0], axis=1, keepdims=True)
        oh = stacks[0] == kmin
        for r in range(R_DEPTH - 1):
            stacks[r] = jnp.where(oh, stacks[r + 1], stacks[r])
        stacks[R_DEPTH - 1] = jnp.where(oh, inf, stacks[R_DEPTH - 1])
        cols.append(jnp.bitwise_and(kmin, jnp.int32(0xFFF)))
    idx_ref[...] = jnp.concatenate(cols, axis=1)


def _sc_combine_body(table_hbm, idx_hbm, out_hbm, tab_v, idx_v, acc_v):
    wid = lax.axis_index("s") * 2 + lax.axis_index("c")
    b = wid // 16
    base = (wid % 16) * QW
    pltpu.sync_copy(table_hbm.at[b], tab_v)
    pltpu.sync_copy(idx_hbm.at[b, pl.ds(base * K, QW * K)], idx_v)
    iota16 = lax.broadcasted_iota(jnp.int32, (16,), 0)

    def body_qb(qb, acc):
        cent = [tab_v[pl.ds(c * N + base + qb * 16, 16)] for c in range(16)]

        def body_k(k, acc):
            ivec = plsc.load_gather(idx_v, [qb * 256 + k + iota16 * K])
            g = [plsc.load_gather(tab_v, [ivec + c * N]) for c in range(16)]
            dr = g[0] - cent[0]
            dg = g[1] - cent[1]
            db = g[2] - cent[2]
            d2 = dr * dr + dg * dg + db * db + 1e-12
            i = lax.bitcast_convert_type(d2, jnp.int32)
            i = jnp.int32(0x5F3759DF) - lax.shift_right_arithmetic(i, 1)
            y = lax.bitcast_convert_type(i, jnp.float32)
            for _ in range(3):
                y = y * (1.5 - 0.5 * d2 * y * y)
            rd = d2 * y
            w = jnp.exp(-10.0 * rd)
            pd = None
            for c in range(3, 16):
                t = jnp.abs(g[c] - cent[c])
                pd = t if pd is None else pd + t
            return acc + w * pd

        return lax.fori_loop(0, 16, body_k, acc)

    acc = lax.fori_loop(0, 16, body_qb, jnp.zeros((16,), jnp.float32))
    acc_v[...] = acc
    pltpu.sync_copy(acc_v, out_hbm.at[wid])


@functools.lru_cache(maxsize=1)
def _get_sc_combine():
    return functools.partial(
        pl.kernel,
        mesh=plsc.VectorSubcoreMesh(core_axis_name="c", subcore_axis_name="s"),
        out_type=jax.ShapeDtypeStruct((NW, 16), jnp.float32),
        scratch_types=[
            pltpu.VMEM((16 * N,), jnp.float32),
            pltpu.VMEM((QW * K,), jnp.int32),
            pltpu.VMEM((16,), jnp.float32),
        ],
        compiler_params=pltpu.CompilerParams(needs_layout_passes=False),
    )(_sc_combine_body)


def kernel(logits, xyz, rgb):
    B = logits.shape[0]
    xyzT = jnp.transpose(xyz, (0, 2, 1))

    table = pl.pallas_call(
        _table_kernel,
        grid=(B,),
        in_specs=[
            pl.BlockSpec((None, C, N), lambda b: (b, 0, 0)),
            pl.BlockSpec((None, 3, N), lambda b: (b, 0, 0)),
        ],
        out_shape=jax.ShapeDtypeStruct((B, 16, N), jnp.float32),
        out_specs=pl.BlockSpec((None, 16, N), lambda b: (b, 0, 0)),
    )(logits, rgb)

    nsteps = N // QB
    idx = pl.pallas_call(
        _topk_kernel,
        grid=(B, nsteps),
        in_specs=[
            pl.BlockSpec((None, QB, 3), lambda b, i: (b, i, 0)),
            pl.BlockSpec((None, 3, N), lambda b, i: (b, 0, 0)),
        ],
        out_shape=jax.ShapeDtypeStruct((B, N, K), jnp.int32),
        out_specs=pl.BlockSpec((None, QB, K), lambda b, i: (b, i, 0)),
    )(xyzT, xyz)

    partials = _get_sc_combine()(table.reshape(B, 16 * N),
                                 idx.reshape(B, N * K))
    return jnp.sum(partials) / (B * N * K)


# speedup vs baseline: 53.5936x; 1.4255x over previous
import functools

import jax
import jax.numpy as jnp
from jax import lax
from jax.experimental import pallas as pl
from jax.experimental.pallas import tpu as pltpu
from jax.experimental.pallas import tpu_sc as plsc

K = 16
N = 4096
C = 13
QB = 256
NW = 32
QW = 256


def _table_kernel(logits_ref, rgb_ref, table_ref):
    x = logits_ref[...]
    m = jnp.max(x, axis=0, keepdims=True)
    e = jnp.exp(x - m)
    s = jnp.sum(e, axis=0, keepdims=True)
    table_ref[0:3, :] = rgb_ref[...]
    table_ref[3:16, :] = e / s


R_DEPTH = 4
NLB = N // 128


def _topk_kernel(xyzq_ref, xyzall_ref, idx_ref):
    xq = xyzq_ref[...]
    xall = xyzall_ref[...]
    ab = jnp.dot(xq.astype(jnp.bfloat16), xall.astype(jnp.bfloat16),
                 preferred_element_type=jnp.float32)
    xxq = jnp.sum(xq * xq, axis=1, keepdims=True)
    xxa = jnp.sum(xall * xall, axis=0, keepdims=True)
    big = jnp.float32(3.0e38)
    iota = lax.broadcasted_iota(jnp.int32, (QB, 128), 1)
    stacks = [jnp.full((QB, 128), big, jnp.float32) for _ in range(R_DEPTH)]
    for c in range(NLB):
        sl = slice(c * 128, (c + 1) * 128)
        d = (xxq + (-2.0) * ab[:, sl]) + xxa[:, sl]
        d = jnp.maximum(d, 0.0)
        kb = jnp.bitwise_and(lax.bitcast_convert_type(d, jnp.int32),
                             jnp.int32(-4096))
        v = lax.bitcast_convert_type(
            kb + (iota + jnp.int32(c * 128 + 0x00800000)), jnp.float32)
        for r in range(R_DEPTH):
            lo = jnp.minimum(stacks[r], v)
            v = jnp.maximum(stacks[r], v)
            stacks[r] = lo
    cols = []
    for _ in range(K):
        kmin = jnp.min(stacks[0], axis=1, keepdims=True)
        oh = stacks[0] == kmin
        for r in range(R_DEPTH - 1):
            stacks[r] = jnp.where(oh, stacks[r + 1], stacks[r])
        stacks[R_DEPTH - 1] = jnp.where(oh, big, stacks[R_DEPTH - 1])
        cols.append(jnp.bitwise_and(
            lax.bitcast_convert_type(kmin, jnp.int32), jnp.int32(0xFFF)))
    idx_ref[...] = jnp.concatenate(cols, axis=1)


def _sc_combine_body(table_hbm, idx_hbm, out_hbm, tab_v, idx_v, acc_v):
    wid = lax.axis_index("s") * 2 + lax.axis_index("c")
    b = wid // 16
    base = (wid % 16) * QW
    pltpu.sync_copy(table_hbm.at[b], tab_v)
    pltpu.sync_copy(idx_hbm.at[b, pl.ds(base * K, QW * K)], idx_v)
    iota16 = lax.broadcasted_iota(jnp.int32, (16,), 0)

    def body_qb(qb, acc):
        cent = [tab_v[pl.ds(c * N + base + qb * 16, 16)] for c in range(16)]

        def body_k(k, acc):
            ivec = plsc.load_gather(idx_v, [qb * 256 + k + iota16 * K])
            g = [plsc.load_gather(tab_v, [ivec + c * N]) for c in range(16)]
            dr = g[0] - cent[0]
            dg = g[1] - cent[1]
            db = g[2] - cent[2]
            d2 = dr * dr + dg * dg + db * db + 1e-12
            i = lax.bitcast_convert_type(d2, jnp.int32)
            i = jnp.int32(0x5F3759DF) - lax.shift_right_arithmetic(i, 1)
            y = lax.bitcast_convert_type(i, jnp.float32)
            for _ in range(3):
                y = y * (1.5 - 0.5 * d2 * y * y)
            rd = d2 * y
            w = jnp.exp(-10.0 * rd)
            pd = None
            for c in range(3, 16):
                t = jnp.abs(g[c] - cent[c])
                pd = t if pd is None else pd + t
            return acc + w * pd

        return lax.fori_loop(0, 16, body_k, acc)

    acc = lax.fori_loop(0, 16, body_qb, jnp.zeros((16,), jnp.float32))
    acc_v[...] = acc
    pltpu.sync_copy(acc_v, out_hbm.at[wid])


@functools.lru_cache(maxsize=1)
def _get_sc_combine():
    return functools.partial(
        pl.kernel,
        mesh=plsc.VectorSubcoreMesh(core_axis_name="c", subcore_axis_name="s"),
        out_type=jax.ShapeDtypeStruct((NW, 16), jnp.float32),
        scratch_types=[
            pltpu.VMEM((16 * N,), jnp.float32),
            pltpu.VMEM((QW * K,), jnp.int32),
            pltpu.VMEM((16,), jnp.float32),
        ],
        compiler_params=pltpu.CompilerParams(needs_layout_passes=False),
    )(_sc_combine_body)


def kernel(logits, xyz, rgb):
    B = logits.shape[0]
    xyzT = jnp.transpose(xyz, (0, 2, 1))

    table = pl.pallas_call(
        _table_kernel,
        grid=(B,),
        in_specs=[
            pl.BlockSpec((None, C, N), lambda b: (b, 0, 0)),
            pl.BlockSpec((None, 3, N), lambda b: (b, 0, 0)),
        ],
        out_shape=jax.ShapeDtypeStruct((B, 16, N), jnp.float32),
        out_specs=pl.BlockSpec((None, 16, N), lambda b: (b, 0, 0)),
    )(logits, rgb)

    nsteps = N // QB
    idx = pl.pallas_call(
        _topk_kernel,
        grid=(B, nsteps),
        in_specs=[
            pl.BlockSpec((None, QB, 3), lambda b, i: (b, i, 0)),
            pl.BlockSpec((None, 3, N), lambda b, i: (b, 0, 0)),
        ],
        out_shape=jax.ShapeDtypeStruct((B, N, K), jnp.int32),
        out_specs=pl.BlockSpec((None, QB, K), lambda b, i: (b, i, 0)),
    )(xyzT, xyz)

    partials = _get_sc_combine()(table.reshape(B, 16 * N),
                                 idx.reshape(B, N * K))
    return jnp.sum(partials) / (B * N * K)
